# Initial kernel scaffold; baseline (speedup 1.0000x reference)
import jax
import jax.numpy as jnp
from jax.experimental import pallas as pl


def kernel(x, x_y_index, W1, a1, W2, a2, p1, p2, lw1, lb1, lw2, lb2, edge_index, node_type, tree):
    raise NotImplementedError("write your pallas kernel here")


# speedup vs baseline: 16.6978x; 16.6978x over previous
import functools

import jax
import jax.numpy as jnp
from jax import lax
from jax.experimental import pallas as pl
from jax.experimental.pallas import tpu as pltpu
from jax.experimental.pallas import tpu_sc as plsc

N = 10000
NPAD = 10112
D = 128
TW = 144
E = 320000
NC, NS, CH = 2, 16, 64
NW = NC * NS
EPT = 10240
NCHUNK = EPT // CH
EPAD = NW * EPT
RPT = NPAD // NS
PADNODE = N + 50
BLK = 128
NBLK = NPAD // BLK

_f32 = jnp.float32


def _ln_rows(x):
    m = jnp.mean(x, axis=-1, keepdims=True)
    v = jnp.mean((x - m) ** 2, axis=-1, keepdims=True)
    return (x - m) * lax.rsqrt(v + 1e-5)


def _stage1_body(x_ref, w1_ref, a1t_ref, nt_ref, tab_ref):
    i = pl.program_id(0)
    h0 = _ln_rows(x_ref[...])
    h1 = jnp.dot(h0, w1_ref[...], preferred_element_type=_f32)
    ev = jnp.dot(h1, a1t_ref[...], preferred_element_type=_f32)
    nt = nt_ref[...]
    oh = (lax.broadcasted_iota(jnp.int32, (BLK, 8), 1) == nt).astype(_f32)
    t = jnp.sum(ev * oh, axis=-1, keepdims=True)
    rid = i * BLK + lax.broadcasted_iota(jnp.int32, (BLK, 1), 0)
    w = jnp.where(rid < N, jnp.exp(t), 0.0)
    tab_ref[...] = jnp.concatenate(
        [w * h1, w, jnp.zeros((BLK, TW - D - 1), _f32)], axis=1)


def _stage1(xp, W1, a1t, ntp):
    return pl.pallas_call(
        _stage1_body,
        grid=(NBLK,),
        in_specs=[
            pl.BlockSpec((BLK, D), lambda i: (i, 0)),
            pl.BlockSpec((D, D), lambda i: (0, 0)),
            pl.BlockSpec((D, 8), lambda i: (0, 0)),
            pl.BlockSpec((BLK, 1), lambda i: (i, 0)),
        ],
        out_specs=pl.BlockSpec((BLK, TW), lambda i: (i, 0)),
        out_shape=jax.ShapeDtypeStruct((NPAD, TW), _f32),
    )(xp, W1, a1t, ntp)


def _stage2_body(has_mask, *args):
    if has_mask:
        p0_ref, p1_ref, pv_ref, m_ref, h_ref, sraw_ref, ssel_ref = args
    else:
        p0_ref, p1_ref, pv_ref, h_ref, sraw_ref, ssel_ref = args
    i = pl.program_id(0)
    s = p0_ref[0] + p1_ref[0]
    num = s[:, :D]
    den = s[:, D:D + 1]
    r = jnp.maximum(num / (den + 1e-16), 0.0)
    h = _ln_rows(r)
    sc = jnp.tanh(jnp.dot(h, pv_ref[...], preferred_element_type=_f32))[:, 0:1]
    h_ref[...] = h
    sraw_ref[...] = sc
    if has_mask:
        valid = m_ref[...] > 0
    else:
        rid = i * BLK + lax.broadcasted_iota(jnp.int32, (BLK, 1), 0)
        valid = rid < N
    ssel_ref[...] = jnp.where(valid, sc, -jnp.inf)


def _stage2(parts, pv, mcol=None):
    has_mask = mcol is not None
    in_specs = [
        pl.BlockSpec((1, BLK, TW), lambda i: (0, i, 0)),
        pl.BlockSpec((1, BLK, TW), lambda i: (1, i, 0)),
        pl.BlockSpec((D, 8), lambda i: (0, 0)),
    ]
    ops = (parts, parts, pv)
    if has_mask:
        in_specs.append(pl.BlockSpec((BLK, 1), lambda i: (i, 0)))
        ops = ops + (mcol,)
    return pl.pallas_call(
        functools.partial(_stage2_body, has_mask),
        grid=(NBLK,),
        in_specs=in_specs,
        out_specs=[
            pl.BlockSpec((BLK, D), lambda i: (i, 0)),
            pl.BlockSpec((BLK, 1), lambda i: (i, 0)),
            pl.BlockSpec((BLK, 1), lambda i: (i, 0)),
        ],
        out_shape=[
            jax.ShapeDtypeStruct((NPAD, D), _f32),
            jax.ShapeDtypeStruct((NPAD, 1), _f32),
            jax.ShapeDtypeStruct((NPAD, 1), _f32),
        ],
    )(*ops)


def _select_body(k, s_ref, mask_ref):
    s = s_ref[...]
    ik = lax.bitcast_convert_type(s, jnp.int32)
    skey = jnp.where(ik >= 0, ik, ik ^ jnp.int32(0x7FFFFFFF))
    lo = jnp.min(skey)
    hi = jnp.max(skey) + 1

    def it(_, c):
        l, h = c
        mid = (l >> 1) + (h >> 1) + (l & h & 1)
        big = jnp.sum((skey >= mid).astype(jnp.int32)) >= k
        return (jnp.where(big, mid, l), jnp.where(big, h, mid))

    thr, _ = lax.fori_loop(0, 32, it, (lo, hi))
    gt = skey > thr
    eq = skey == thr
    need = (k - jnp.sum(gt.astype(jnp.int32))).astype(_f32)
    eqf = eq.astype(_f32)
    a0 = lax.broadcasted_iota(jnp.int32, (BLK, BLK), 0)
    a1 = lax.broadcasted_iota(jnp.int32, (BLK, BLK), 1)
    mtri = (a0 <= a1).astype(_f32)
    pref = jnp.dot(eqf, mtri, preferred_element_type=_f32)
    rt = jnp.sum(eqf, axis=-1, keepdims=True)
    b0 = lax.broadcasted_iota(jnp.int32, (NBLK, NBLK), 0)
    b1 = lax.broadcasted_iota(jnp.int32, (NBLK, NBLK), 1)
    ltri = (b1 < b0).astype(_f32)
    rex = jnp.dot(ltri, rt, preferred_element_type=_f32)
    cum = rex + pref
    sel = jnp.logical_or(gt, jnp.logical_and(eq, cum <= need))
    mask_ref[...] = sel.astype(_f32)


def _select(ssel, k):
    return pl.pallas_call(
        functools.partial(_select_body, k),
        in_specs=[pl.BlockSpec((NBLK, BLK), lambda: (0, 0))],
        out_specs=pl.BlockSpec((NBLK, BLK), lambda: (0, 0)),
        out_shape=jax.ShapeDtypeStruct((NBLK, BLK), _f32),
    )(ssel)


def _build2_body(h_ref, s_ref, m_ref, w2_ref, a2t_ref, nt_ref, tab_ref, x1_ref):
    i = pl.program_id(0)

    @pl.when(i == 0)
    def _():
        x1_ref[...] = jnp.zeros_like(x1_ref)

    h = h_ref[...]
    hp = h * s_ref[...]
    mk = m_ref[...]
    x1_ref[...] += jnp.sum(hp * mk, axis=0, keepdims=True)
    h2 = jnp.dot(hp, w2_ref[...], preferred_element_type=_f32)
    ev = jnp.dot(h2, a2t_ref[...], preferred_element_type=_f32)
    oh = (lax.broadcasted_iota(jnp.int32, (BLK, 8), 1) == nt_ref[...]).astype(_f32)
    t = jnp.sum(ev * oh, axis=-1, keepdims=True)
    w = jnp.exp(t) * mk
    tab_ref[...] = jnp.concatenate(
        [w * h2, w, jnp.zeros((BLK, TW - D - 1), _f32)], axis=1)


def _build2(h, s1raw, m1col, W2, a2t, ntp):
    return pl.pallas_call(
        _build2_body,
        grid=(NBLK,),
        in_specs=[
            pl.BlockSpec((BLK, D), lambda i: (i, 0)),
            pl.BlockSpec((BLK, 1), lambda i: (i, 0)),
            pl.BlockSpec((BLK, 1), lambda i: (i, 0)),
            pl.BlockSpec((D, D), lambda i: (0, 0)),
            pl.BlockSpec((D, 8), lambda i: (0, 0)),
            pl.BlockSpec((BLK, 1), lambda i: (i, 0)),
        ],
        out_specs=[
            pl.BlockSpec((BLK, TW), lambda i: (i, 0)),
            pl.BlockSpec((1, D), lambda i: (0, 0)),
        ],
        out_shape=[
            jax.ShapeDtypeStruct((NPAD, TW), _f32),
            jax.ShapeDtypeStruct((1, D), _f32),
        ],
    )(h, s1raw, m1col, W2, a2t, ntp)


def _x2_body(h_ref, s_ref, m_ref, x2_ref):
    i = pl.program_id(0)

    @pl.when(i == 0)
    def _():
        x2_ref[...] = jnp.zeros_like(x2_ref)

    x2_ref[...] += jnp.sum(h_ref[...] * s_ref[...] * m_ref[...],
                           axis=0, keepdims=True)


def _x2(h3, s2raw, m2col):
    return pl.pallas_call(
        _x2_body,
        grid=(NBLK,),
        in_specs=[
            pl.BlockSpec((BLK, D), lambda i: (i, 0)),
            pl.BlockSpec((BLK, 1), lambda i: (i, 0)),
            pl.BlockSpec((BLK, 1), lambda i: (i, 0)),
        ],
        out_specs=pl.BlockSpec((1, D), lambda i: (0, 0)),
        out_shape=jax.ShapeDtypeStruct((1, D), _f32),
    )(h3, s2raw, m2col)


def _head_body(x1_ref, x2_ref, lw1_ref, lb1_ref, lw2_ref, lb2_ref, o_ref):
    g = x1_ref[...] / 2000.0 + x2_ref[...] / 500.0
    g1 = jnp.dot(g, lw1_ref[...], preferred_element_type=_f32) + lb1_ref[...]
    g1 = jnp.maximum(g1, 0.0)
    lane = lax.broadcasted_iota(jnp.int32, (1, BLK), 1)
    m64 = (lane < 64).astype(_f32)
    mu = jnp.sum(g1 * m64) / 64.0
    va = jnp.sum(((g1 - mu) * m64) ** 2) / 64.0
    z = (g1 - mu) * lax.rsqrt(va + 1e-5) * m64
    o = jnp.dot(z, lw2_ref[...], preferred_element_type=_f32) + lb2_ref[...]
    om = jnp.where(lane < 6, o, -jnp.inf)
    e = jnp.exp(om - jnp.max(om))
    o_ref[...] = e / jnp.sum(e)


def _head(x1p, x2p, lw1p, lb1p, lw2p, lb2p):
    full = pl.BlockSpec((D, D), lambda: (0, 0))
    row = pl.BlockSpec((1, D), lambda: (0, 0))
    return pl.pallas_call(
        _head_body,
        in_specs=[row, row, full, row, full, row],
        out_specs=row,
        out_shape=jax.ShapeDtypeStruct((1, D), _f32),
    )(x1p, x2p, lw1p, lb1p, lw2p, lb2p)


def _conv_sc_body(tab_hbm, src_hbm, dst_hbm, zero_hbm, out_hbm,
                  src_v, dst_v, rows0, rows1, acc, sem0, sem1):
    cid = lax.axis_index("c")
    sid = lax.axis_index("s")
    wid = cid * NS + sid
    pltpu.sync_copy(zero_hbm, acc.at[pl.ds(sid * RPT, RPT)])
    plsc.subcore_barrier()
    pltpu.sync_copy(src_hbm.at[wid], src_v)
    pltpu.sync_copy(dst_hbm.at[wid], dst_v)
    pltpu.async_copy(tab_hbm.at[src_v.at[0]], rows0, sem0)

    def body(kk, carry):
        j0 = kk * 2
        j1 = j0 + 1
        pltpu.async_copy(tab_hbm.at[src_v.at[j1]], rows1, sem1)
        pltpu.make_async_copy(tab_hbm.at[src_v.at[j0]], rows0, sem0).wait()
        pltpu.sync_copy(rows0, acc.at[dst_v.at[j0]], add=True)

        @pl.when(kk < NCHUNK // 2 - 1)
        def _():
            pltpu.async_copy(tab_hbm.at[src_v.at[j0 + 2]], rows0, sem0)

        pltpu.make_async_copy(tab_hbm.at[src_v.at[j1]], rows1, sem1).wait()
        pltpu.sync_copy(rows1, acc.at[dst_v.at[j1]], add=True)
        return carry

    lax.fori_loop(0, NCHUNK // 2, body, 0)
    plsc.subcore_barrier()
    pltpu.sync_copy(acc.at[pl.ds(sid * RPT, RPT)],
                    out_hbm.at[cid, pl.ds(sid * RPT, RPT)])


@functools.lru_cache(maxsize=1)
def _conv_sc_call():
    return pl.kernel(
        _conv_sc_body,
        out_type=jax.ShapeDtypeStruct((NC, NPAD, TW), _f32),
        mesh=plsc.VectorSubcoreMesh(core_axis_name="c", subcore_axis_name="s",
                                    num_cores=NC, num_subcores=NS),
        scratch_types=[
            pltpu.VMEM((NCHUNK, CH), jnp.int32),
            pltpu.VMEM((NCHUNK, CH), jnp.int32),
            pltpu.VMEM((CH, TW), _f32),
            pltpu.VMEM((CH, TW), _f32),
            pltpu.VMEM_SHARED((NPAD, TW), _f32),
            pltpu.SemaphoreType.DMA,
            pltpu.SemaphoreType.DMA,
        ],
        compiler_params=pltpu.CompilerParams(use_tc_tiling_on_sc=False),
    )


def _conv_accumulate(tab, src_r, dst_r, zeros_blk):
    return _conv_sc_call()(tab, src_r, dst_r, zeros_blk)


def kernel(x, x_y_index, W1, a1, W2, a2, p1, p2, lw1, lb1, lw2, lb2,
           edge_index, node_type, tree):
    xp = jnp.zeros((NPAD, D), _f32).at[:N].set(x)
    ntp = jnp.zeros((NPAD, 1), jnp.int32).at[:N, 0].set(node_type)
    a1t = jnp.zeros((D, 8), _f32).at[:, :3].set(a1.T)
    a2t = jnp.zeros((D, 8), _f32).at[:, :3].set(a2.T)
    p1v = jnp.zeros((D, 8), _f32).at[:, 0].set(p1)
    p2v = jnp.zeros((D, 8), _f32).at[:, 0].set(p2)
    lw1p = jnp.zeros((D, D), _f32).at[:, :64].set(lw1)
    lb1p = jnp.zeros((1, D), _f32).at[0, :64].set(lb1)
    lw2p = jnp.zeros((D, D), _f32).at[:64, :6].set(lw2)
    lb2p = jnp.zeros((1, D), _f32).at[0, :6].set(lb2)
    ei = jnp.full((2, EPAD), PADNODE, jnp.int32).at[:, :E].set(edge_index)
    src_r = ei[0].reshape(NW, NCHUNK, CH)
    dst_r = ei[1].reshape(NW, NCHUNK, CH)
    zeros_blk = jnp.zeros((RPT, TW), _f32)

    tab1 = _stage1(xp, W1, a1t, ntp)
    parts1 = _conv_accumulate(tab1, src_r, dst_r, zeros_blk)
    h, s1raw, s1sel = _stage2(parts1, p1v)
    mask1 = _select(s1sel.reshape(NBLK, BLK), 2000)
    m1col = mask1.reshape(NPAD, 1)
    tab2, x1p = _build2(h, s1raw, m1col, W2, a2t, ntp)
    parts2 = _conv_accumulate(tab2, src_r, dst_r, zeros_blk)
    h3, s2raw, s2sel = _stage2(parts2, p2v, m1col)
    mask2 = _select(s2sel.reshape(NBLK, BLK), 500)
    x2p = _x2(h3, s2raw, mask2.reshape(NPAD, 1))
    o = _head(x1p, x2p, lw1p, lb1p, lw2p, lb2p)
    return o[:, :6]
